# Initial kernel scaffold; baseline (speedup 1.0000x reference)
import jax
import jax.numpy as jnp
from jax.experimental import pallas as pl


def kernel(x, edge_index, edge_attr, W1, b1, W2, b2):
    raise NotImplementedError("write your pallas kernel here")


# speedup vs baseline: 3.6965x; 3.6965x over previous
import functools
import jax
import jax.numpy as jnp
from jax import lax
from jax.experimental import pallas as pl
from jax.experimental.pallas import tpu as pltpu
from jax.experimental.pallas import tpu_sc as plsc

N = 10000
E = 160000
D = 256
H = 128
NS = 16
EC = E // NS
K = 80
NCH = EC // K
ED = E // (2 * NS)
KD = 40
NCHD = ED // KD

_mesh = plsc.VectorSubcoreMesh(core_axis_name="c", subcore_axis_name="s")


def _deg_body(dst_h, w_h, zer_h, out_h, dstv, wv, deg_sp):
    c = lax.axis_index("c")
    s = lax.axis_index("s")

    @pl.when(s == 0)
    def _():
        pltpu.sync_copy(zer_h, deg_sp)

    plsc.subcore_barrier()
    pltpu.sync_copy(dst_h.at[c, s], dstv)
    pltpu.sync_copy(w_h.at[c, s], wv)

    def chunk(j, carry):
        pltpu.sync_copy(wv.at[j], deg_sp.at[dstv.at[j]], add=True)
        return carry

    lax.fori_loop(0, NCHD, chunk, 0)
    plsc.subcore_barrier()

    @pl.when(s == 0)
    def _():
        pltpu.sync_copy(deg_sp, out_h.at[c])


_deg_call = pl.kernel(
    _deg_body,
    out_type=jax.ShapeDtypeStruct((2, N), jnp.float32),
    mesh=_mesh,
    scratch_types=[
        pltpu.VMEM((NCHD, KD), jnp.int32),
        pltpu.VMEM((NCHD, KD), jnp.float32),
        pltpu.VMEM_SHARED((N,), jnp.float32),
    ],
)


NP = 3
NH = 3336
RWB = 208
RST = 104


def _scatter_body(p_h, srclo_h, srchi_h, dstm_h, wm_h, out_h,
                  idxv, dstm, wp, rows0, rows1, stage, acc_sp):
    c = lax.axis_index("c")
    s = lax.axis_index("s")

    @pl.when(c == 0)
    def _():
        pltpu.sync_copy(srclo_h.at[s], idxv)

    @pl.when(c == 1)
    def _():
        pltpu.sync_copy(srchi_h.at[s], idxv)

    pltpu.sync_copy(dstm_h.at[s], dstm)

    def zstep(i, carry):
        stage[i // 8, pl.ds((i % 8) * 16, 16)] = jnp.zeros((16,), jnp.float32)
        return carry

    lax.fori_loop(0, RST * H // 16, zstep, 0)

    def scale(rows, j):
        def estep(eb, carry):
            w16 = wp[j, pl.ds(eb * 16, 16)]
            for l in range(16):
                e = eb * 16 + l
                w = w16[l]
                for g in range(H // 16):
                    rows[e, pl.ds(g * 16, 16)] = rows[e, pl.ds(g * 16, 16)] * w
            return carry
        lax.fori_loop(0, K // 16, estep, 0)

    for h in range(NP):
        for i in range(RWB // RST):
            pltpu.sync_copy(stage, acc_sp.at[pl.ds(s * RWB + i * RST, RST)])

        @pl.when(s == NS - 1)
        def _():
            pltpu.sync_copy(stage.at[pl.ds(0, NH - NS * RWB)],
                            acc_sp.at[pl.ds(NS * RWB, NH - NS * RWB)])

        pltpu.sync_copy(wm_h.at[h, s], wp)
        plsc.subcore_barrier()

        def body(j, carry):
            pltpu.sync_copy(p_h.at[idxv.at[j]], rows0)
            scale(rows0, j)
            pltpu.sync_copy(rows0, acc_sp.at[dstm.at[j]], add=True)
            return carry

        lax.fori_loop(0, NCH, body, 0)

        plsc.subcore_barrier()
        pltpu.sync_copy(acc_sp.at[pl.ds(s * RWB, RWB)],
                        out_h.at[c, h, pl.ds(s * RWB, RWB)])

        @pl.when(s == NS - 1)
        def _():
            pltpu.sync_copy(acc_sp.at[pl.ds(NS * RWB, NH - NS * RWB)],
                            out_h.at[c, h, pl.ds(NS * RWB, NH - NS * RWB)])

        plsc.subcore_barrier()


_scatter_call = pl.kernel(
    _scatter_body,
    out_type=jax.ShapeDtypeStruct((2, NP, NH, H), jnp.float32),
    mesh=_mesh,
    scratch_types=[
        pltpu.VMEM((NCH, K), jnp.int32),
        pltpu.VMEM((NCH, K), jnp.int32),
        pltpu.VMEM((NCH, K), jnp.float32),
        pltpu.VMEM((K, H), jnp.float32),
        pltpu.VMEM((K, H), jnp.float32),
        pltpu.VMEM((RST, H), jnp.float32),
        pltpu.VMEM_SHARED((NH, H), jnp.float32),
    ],
)


_R = 1000


def _tc1_body(d0, d1, x, w, dinv_o, p_o):
    deg = d0[...] + d1[...] + 1.0
    di = jnp.where(deg > 0, lax.rsqrt(deg), 0.0)
    dinv_o[...] = di
    p_o[...] = jnp.dot(x[...], w[...], preferred_element_type=jnp.float32) * di


_tc1 = pl.pallas_call(
    _tc1_body,
    grid=(N // _R,),
    in_specs=[
        pl.BlockSpec((_R, 1), lambda i: (i, 0)),
        pl.BlockSpec((_R, 1), lambda i: (i, 0)),
        pl.BlockSpec((_R, D), lambda i: (i, 0)),
        pl.BlockSpec((D, D), lambda i: (0, 0)),
    ],
    out_specs=[
        pl.BlockSpec((_R, 1), lambda i: (i, 0)),
        pl.BlockSpec((_R, D), lambda i: (i, 0)),
    ],
    out_shape=[
        jax.ShapeDtypeStruct((N, 1), jnp.float32),
        jax.ShapeDtypeStruct((N, D), jnp.float32),
    ],
)


def _tc2_body(a0, a1, p1, dinv, b, w, p2_o):
    acc = jnp.concatenate([a0[...], a1[...]], axis=1)
    di = dinv[...]
    t = jnp.maximum(di * (acc + p1[...]) + b[...], 0.0)
    p2_o[...] = jnp.dot(t, w[...], preferred_element_type=jnp.float32) * di


_tc2 = pl.pallas_call(
    _tc2_body,
    grid=(N // _R,),
    in_specs=[
        pl.BlockSpec((_R, H), lambda i: (i, 0)),
        pl.BlockSpec((_R, H), lambda i: (i, 0)),
        pl.BlockSpec((_R, D), lambda i: (i, 0)),
        pl.BlockSpec((_R, 1), lambda i: (i, 0)),
        pl.BlockSpec((1, D), lambda i: (0, 0)),
        pl.BlockSpec((D, D), lambda i: (0, 0)),
    ],
    out_specs=pl.BlockSpec((_R, D), lambda i: (i, 0)),
    out_shape=jax.ShapeDtypeStruct((N, D), jnp.float32),
)


def _tc3_body(a0, a1, p2, dinv, b, out_o):
    acc = jnp.concatenate([a0[...], a1[...]], axis=1)
    out_o[...] = dinv[...] * (acc + p2[...]) + b[...]


_tc3 = pl.pallas_call(
    _tc3_body,
    grid=(N // _R,),
    in_specs=[
        pl.BlockSpec((_R, H), lambda i: (i, 0)),
        pl.BlockSpec((_R, H), lambda i: (i, 0)),
        pl.BlockSpec((_R, D), lambda i: (i, 0)),
        pl.BlockSpec((_R, 1), lambda i: (i, 0)),
        pl.BlockSpec((1, D), lambda i: (0, 0)),
    ],
    out_specs=pl.BlockSpec((_R, D), lambda i: (i, 0)),
    out_shape=jax.ShapeDtypeStruct((N, D), jnp.float32),
)


@jax.jit
def kernel(x, edge_index, edge_attr, W1, b1, W2, b2):
    src = edge_index[0]
    dst = edge_index[1]

    degp = _deg_call(
        dst.reshape(2, NS, NCHD, KD),
        edge_attr.reshape(2, NS, NCHD, KD),
        jnp.zeros((N,), jnp.float32),
    )
    dinv, p1 = _tc1(degp[0].reshape(N, 1), degp[1].reshape(N, 1), x, W1)

    srclo = (src * 2).reshape(NS, NCH, K)
    srchi = (src * 2 + 1).reshape(NS, NCH, K)
    bucket = dst // NH
    dstmr = (dst - bucket * NH).reshape(NS, NCH, K)
    wm = jnp.where(bucket[None, :] == jnp.arange(NP, dtype=jnp.int32)[:, None],
                   edge_attr[None, :], 0.0).reshape(NP, NS, NCH, K)

    acc1 = _scatter_call(p1.reshape(2 * N, H), srclo, srchi, dstmr,
                         wm).reshape(2, NP * NH, H)[:, :N, :]
    p2 = _tc2(acc1[0], acc1[1], p1, dinv, b1.reshape(1, D), W2)
    acc2 = _scatter_call(p2.reshape(2 * N, H), srclo, srchi, dstmr,
                         wm).reshape(2, NP * NH, H)[:, :N, :]
    out = _tc3(acc2[0], acc2[1], p2, dinv, b2.reshape(1, D))
    return out


# speedup vs baseline: 5.8126x; 1.5724x over previous
import functools
import jax
import jax.numpy as jnp
from jax import lax
from jax.experimental import pallas as pl
from jax.experimental.pallas import tpu as pltpu
from jax.experimental.pallas import tpu_sc as plsc

N = 10000
E = 160000
D = 256
H = 128
NS = 16
EC = E // NS
K = 80
NCH = EC // K
ED = E // (2 * NS)
KD = 40
NCHD = ED // KD

_mesh = plsc.VectorSubcoreMesh(core_axis_name="c", subcore_axis_name="s")


def _deg_body(dst_h, w_h, zer_h, out_h, dstv, wv, deg_sp):
    c = lax.axis_index("c")
    s = lax.axis_index("s")

    @pl.when(s == 0)
    def _():
        pltpu.sync_copy(zer_h, deg_sp)

    plsc.subcore_barrier()
    pltpu.sync_copy(dst_h.at[c, s], dstv)
    pltpu.sync_copy(w_h.at[c, s], wv)

    def chunk(j, carry):
        pltpu.sync_copy(wv.at[j], deg_sp.at[dstv.at[j]], add=True)
        return carry

    lax.fori_loop(0, NCHD, chunk, 0)
    plsc.subcore_barrier()

    @pl.when(s == 0)
    def _():
        pltpu.sync_copy(deg_sp, out_h.at[c])


_deg_call = pl.kernel(
    _deg_body,
    out_type=jax.ShapeDtypeStruct((2, N), jnp.float32),
    mesh=_mesh,
    scratch_types=[
        pltpu.VMEM((NCHD, KD), jnp.int32),
        pltpu.VMEM((NCHD, KD), jnp.float32),
        pltpu.VMEM_SHARED((N,), jnp.float32),
    ],
)


NP = 3
NH = 3336
RWB = 208
RST = 104


def _scatter_body(p_h, srclo_h, srchi_h, dstm_h, wm_h, out_h,
                  idxv, dstm, wp, rows0, rows1, stage,
                  semg0, semg1, sems0, sems1, acc_sp):
    c = lax.axis_index("c")
    s = lax.axis_index("s")

    @pl.when(c == 0)
    def _():
        pltpu.sync_copy(srclo_h.at[s], idxv)

    @pl.when(c == 1)
    def _():
        pltpu.sync_copy(srchi_h.at[s], idxv)

    pltpu.sync_copy(dstm_h.at[s], dstm)

    def zstep(i, carry):
        stage[i // 8, pl.ds((i % 8) * 16, 16)] = jnp.zeros((16,), jnp.float32)
        return carry

    lax.fori_loop(0, RST * H // 16, zstep, 0)

    def scale(rows, j):
        def estep(eb, carry):
            w16 = wp[j, pl.ds(eb * 16, 16)]
            for l in range(16):
                e = eb * 16 + l
                w = w16[l]
                for g in range(H // 16):
                    rows[e, pl.ds(g * 16, 16)] = rows[e, pl.ds(g * 16, 16)] * w
            return carry
        lax.fori_loop(0, K // 16, estep, 0)

    for h in range(NP):
        for i in range(RWB // RST):
            pltpu.sync_copy(stage, acc_sp.at[pl.ds(s * RWB + i * RST, RST)])

        @pl.when(s == NS - 1)
        def _():
            pltpu.sync_copy(stage.at[pl.ds(0, NH - NS * RWB)],
                            acc_sp.at[pl.ds(NS * RWB, NH - NS * RWB)])

        pltpu.sync_copy(wm_h.at[h, s], wp)
        plsc.subcore_barrier()

        def gather_start(j, rows, sem):
            pltpu.async_copy(p_h.at[idxv.at[j]], rows, sem)

        def gather_wait(j, rows, sem):
            pltpu.make_async_copy(p_h.at[idxv.at[j]], rows, sem).wait()

        def scat_start(j, rows, sem):
            pltpu.async_copy(rows, acc_sp.at[dstm.at[j]], sem, add=True)

        def scat_wait(j, rows, sem):
            pltpu.make_async_copy(rows, acc_sp.at[dstm.at[j]], sem).wait()

        gather_start(0, rows0, semg0)

        def body(jj, carry):
            j0 = jj * 2
            j1 = j0 + 1
            gather_wait(j0, rows0, semg0)

            @pl.when(jj > 0)
            def _():
                scat_wait(j0 - 1, rows1, sems1)

            gather_start(j1, rows1, semg1)
            scale(rows0, j0)
            scat_start(j0, rows0, sems0)

            gather_wait(j1, rows1, semg1)
            scat_wait(j0, rows0, sems0)

            @pl.when(jj < NCH // 2 - 1)
            def _():
                gather_start(j0 + 2, rows0, semg0)

            scale(rows1, j1)
            scat_start(j1, rows1, sems1)
            return carry

        lax.fori_loop(0, NCH // 2, body, 0)
        jl = NCH - 1
        gather_start(jl, rows0, semg0)
        gather_wait(jl, rows0, semg0)
        scale(rows0, jl)
        scat_start(jl, rows0, sems0)
        scat_wait(jl, rows0, sems0)
        scat_wait(jl - 1, rows1, sems1)

        plsc.subcore_barrier()
        pltpu.sync_copy(acc_sp.at[pl.ds(s * RWB, RWB)],
                        out_h.at[c, h, pl.ds(s * RWB, RWB)])

        @pl.when(s == NS - 1)
        def _():
            pltpu.sync_copy(acc_sp.at[pl.ds(NS * RWB, NH - NS * RWB)],
                            out_h.at[c, h, pl.ds(NS * RWB, NH - NS * RWB)])

        plsc.subcore_barrier()


_scatter_call = pl.kernel(
    _scatter_body,
    out_type=jax.ShapeDtypeStruct((2, NP, NH, H), jnp.float32),
    mesh=_mesh,
    scratch_types=[
        pltpu.VMEM((NCH, K), jnp.int32),
        pltpu.VMEM((NCH, K), jnp.int32),
        pltpu.VMEM((NCH, K), jnp.float32),
        pltpu.VMEM((K, H), jnp.float32),
        pltpu.VMEM((K, H), jnp.float32),
        pltpu.VMEM((RST, H), jnp.float32),
        pltpu.SemaphoreType.DMA,
        pltpu.SemaphoreType.DMA,
        pltpu.SemaphoreType.DMA,
        pltpu.SemaphoreType.DMA,
        pltpu.VMEM_SHARED((NH, H), jnp.float32),
    ],
)


_R = 1000


def _tc1_body(d0, d1, x, w, dinv_o, p_o):
    deg = d0[...] + d1[...] + 1.0
    di = jnp.where(deg > 0, lax.rsqrt(deg), 0.0)
    dinv_o[...] = di
    p_o[...] = jnp.dot(x[...], w[...], preferred_element_type=jnp.float32) * di


_tc1 = pl.pallas_call(
    _tc1_body,
    grid=(N // _R,),
    in_specs=[
        pl.BlockSpec((_R, 1), lambda i: (i, 0)),
        pl.BlockSpec((_R, 1), lambda i: (i, 0)),
        pl.BlockSpec((_R, D), lambda i: (i, 0)),
        pl.BlockSpec((D, D), lambda i: (0, 0)),
    ],
    out_specs=[
        pl.BlockSpec((_R, 1), lambda i: (i, 0)),
        pl.BlockSpec((_R, D), lambda i: (i, 0)),
    ],
    out_shape=[
        jax.ShapeDtypeStruct((N, 1), jnp.float32),
        jax.ShapeDtypeStruct((N, D), jnp.float32),
    ],
)


def _tc2_body(a0, a1, p1, dinv, b, w, p2_o):
    acc = jnp.concatenate([a0[...], a1[...]], axis=1)
    di = dinv[...]
    t = jnp.maximum(di * (acc + p1[...]) + b[...], 0.0)
    p2_o[...] = jnp.dot(t, w[...], preferred_element_type=jnp.float32) * di


_tc2 = pl.pallas_call(
    _tc2_body,
    grid=(N // _R,),
    in_specs=[
        pl.BlockSpec((_R, H), lambda i: (i, 0)),
        pl.BlockSpec((_R, H), lambda i: (i, 0)),
        pl.BlockSpec((_R, D), lambda i: (i, 0)),
        pl.BlockSpec((_R, 1), lambda i: (i, 0)),
        pl.BlockSpec((1, D), lambda i: (0, 0)),
        pl.BlockSpec((D, D), lambda i: (0, 0)),
    ],
    out_specs=pl.BlockSpec((_R, D), lambda i: (i, 0)),
    out_shape=jax.ShapeDtypeStruct((N, D), jnp.float32),
)


def _tc3_body(a0, a1, p2, dinv, b, out_o):
    acc = jnp.concatenate([a0[...], a1[...]], axis=1)
    out_o[...] = dinv[...] * (acc + p2[...]) + b[...]


_tc3 = pl.pallas_call(
    _tc3_body,
    grid=(N // _R,),
    in_specs=[
        pl.BlockSpec((_R, H), lambda i: (i, 0)),
        pl.BlockSpec((_R, H), lambda i: (i, 0)),
        pl.BlockSpec((_R, D), lambda i: (i, 0)),
        pl.BlockSpec((_R, 1), lambda i: (i, 0)),
        pl.BlockSpec((1, D), lambda i: (0, 0)),
    ],
    out_specs=pl.BlockSpec((_R, D), lambda i: (i, 0)),
    out_shape=jax.ShapeDtypeStruct((N, D), jnp.float32),
)


@jax.jit
def kernel(x, edge_index, edge_attr, W1, b1, W2, b2):
    src = edge_index[0]
    dst = edge_index[1]

    degp = _deg_call(
        dst.reshape(2, NS, NCHD, KD),
        edge_attr.reshape(2, NS, NCHD, KD),
        jnp.zeros((N,), jnp.float32),
    )
    dinv, p1 = _tc1(degp[0].reshape(N, 1), degp[1].reshape(N, 1), x, W1)

    srclo = (src * 2).reshape(NS, NCH, K)
    srchi = (src * 2 + 1).reshape(NS, NCH, K)
    bucket = dst // NH
    dstmr = (dst - bucket * NH).reshape(NS, NCH, K)
    wm = jnp.where(bucket[None, :] == jnp.arange(NP, dtype=jnp.int32)[:, None],
                   edge_attr[None, :], 0.0).reshape(NP, NS, NCH, K)

    acc1 = _scatter_call(p1.reshape(2 * N, H), srclo, srchi, dstmr,
                         wm).reshape(2, NP * NH, H)[:, :N, :]
    p2 = _tc2(acc1[0], acc1[1], p1, dinv, b1.reshape(1, D), W2)
    acc2 = _scatter_call(p2.reshape(2 * N, H), srclo, srchi, dstmr,
                         wm).reshape(2, NP * NH, H)[:, :N, :]
    out = _tc3(acc2[0], acc2[1], p2, dinv, b2.reshape(1, D))
    return out
